# baseline (speedup 1.0000x reference)
import functools

import jax
import jax.numpy as jnp
from jax import lax
from jax.experimental import pallas as pl
from jax.experimental.pallas import tpu as pltpu
from jax.experimental.pallas import tpu_sc as plsc

H = 256
CS = 18
OUT_C = 16

NP = 51200
ROWS = NP * CS
NW = 32
CHUNK = 128
N_CHUNKS_TOT = ROWS // CHUNK
PAIR_CHUNKS = N_CHUNKS_TOT // 16
CORE0_CHUNKS = 129
CORE1_CHUNKS = PAIR_CHUNKS - CORE0_CHUNKS
MAX_CHUNKS = max(CORE0_CHUNKS, CORE1_CHUNKS)
SLAB = ((MAX_CHUNKS + 1 + 7) // 8 + 1) * 8
IDX_ROWS = ((N_CHUNKS_TOT + SLAB + 7) // 8) * 8

NODE_BLK = 400


def _sc_gather_body(x_hbm, idx_hbm, out_hbm, idx_v, buf0, buf1,
                    sem_g0, sem_g1, sem_w0, sem_w1):
    c = lax.axis_index("c")
    s = lax.axis_index("s")
    start = s * PAIR_CHUNKS + jnp.where(c == 0, 0, CORE0_CHUNKS)
    nloc = jnp.where(c == 0, CORE0_CHUNKS, CORE1_CHUNKS)
    astart = pl.multiple_of((start // 8) * 8, 8)
    off = start - astart

    def gather(k, buf, sem):
        pltpu.async_copy(x_hbm.at[idx_v.at[off + k]], buf, sem)

    def write(k, buf, sem):
        pltpu.async_copy(
            buf, out_hbm.at[pl.ds((start + k) * CHUNK, CHUNK)], sem)

    def wait_g(buf, sem, k):
        pltpu.make_async_copy(x_hbm.at[idx_v.at[off + k]], buf, sem).wait()

    def wait_w(buf, sem, k):
        pltpu.make_async_copy(
            buf, out_hbm.at[pl.ds((start + k) * CHUNK, CHUNK)], sem).wait()

    pltpu.sync_copy(idx_hbm.at[pl.ds(astart, SLAB)], idx_v)
    gather(0, buf0, sem_g0)
    wait_g(buf0, sem_g0, 0)
    write(0, buf0, sem_w0)
    gather(1, buf1, sem_g1)

    def body(j, carry):
        a = 2 * j + 1
        wait_g(buf1, sem_g1, a)
        write(a, buf1, sem_w1)
        wait_w(buf0, sem_w0, a - 1)
        gather(a + 1, buf0, sem_g0)
        wait_w(buf1, sem_w1, a)
        gather(a + 2, buf1, sem_g1)
        wait_g(buf0, sem_g0, a + 1)
        write(a + 1, buf0, sem_w0)
        return carry

    lax.fori_loop(0, (nloc - 1) // 2, body, 0)
    wait_g(buf1, sem_g1, nloc)
    wait_w(buf0, sem_w0, nloc - 1)


H2 = H // 2

_sc_gather = functools.partial(
    pl.kernel,
    mesh=plsc.VectorSubcoreMesh(core_axis_name="c", subcore_axis_name="s"),
    out_type=jax.ShapeDtypeStruct((ROWS, H2), jnp.float32),
    scratch_types=[
        pltpu.VMEM((SLAB, CHUNK), jnp.int32),
        pltpu.VMEM((CHUNK, H2), jnp.float32),
        pltpu.VMEM((CHUNK, H2), jnp.float32),
        pltpu.SemaphoreType.DMA,
        pltpu.SemaphoreType.DMA,
        pltpu.SemaphoreType.DMA,
        pltpu.SemaphoreType.DMA,
    ],
)(_sc_gather_body)


def _tc_block(xg_ref, wlo_ref, whi_ref, bs_ref, wf_ref, bf_ref, out_ref):
    u = jax.lax.bitcast_convert_type(xg_ref[...], jnp.uint32)
    xlo = jax.lax.bitcast_convert_type(
        u << 16, jnp.float32).astype(jnp.bfloat16)
    xhi = jax.lax.bitcast_convert_type(
        u & jnp.uint32(0xFFFF0000), jnp.float32).astype(jnp.bfloat16)
    for c in range(OUT_C):
        ylo = jnp.concatenate([xlo[c], xlo[c + 1], xlo[c + 2]], axis=1)
        yhi = jnp.concatenate([xhi[c], xhi[c + 1], xhi[c + 2]], axis=1)
        s = jnp.dot(ylo, wlo_ref[...], preferred_element_type=jnp.float32)
        s = s + jnp.dot(yhi, whi_ref[...], preferred_element_type=jnp.float32)
        s = s + bs_ref[...]
        e = jnp.where(s > 0, s, jnp.exp(jnp.minimum(s, 0.0)) - 1.0)
        o = jnp.dot(e.astype(jnp.bfloat16), wf_ref[...],
                    preferred_element_type=jnp.float32)
        out_ref[:, c, :] = o + bf_ref[...]


def kernel(x, circle_index, W0, b0, W1, b1, W2, b2, Wf, bf):
    n_nodes = x.shape[0]
    ci = circle_index.astype(jnp.int32)
    cit = jnp.pad(ci.T, ((0, 0), (0, NP - n_nodes)))
    idx = jnp.pad(cit.reshape(N_CHUNKS_TOT, CHUNK),
                  ((0, IDX_ROWS - N_CHUNKS_TOT), (0, 0)))

    xp = jax.lax.bitcast_convert_type(
        x.astype(jnp.bfloat16).reshape(n_nodes, H2, 2), jnp.float32)

    xg = _sc_gather(xp, idx)
    xg3 = xg.reshape(CS, NP, H2)

    ws = jnp.concatenate([W0.T, W1.T, W2.T], axis=0)
    ws = ws.reshape(3, H2, 2, H).astype(jnp.bfloat16)
    ws_lo = ws[:, :, 0, :].reshape(3 * H2, H)
    ws_hi = ws[:, :, 1, :].reshape(3 * H2, H)
    bs = (b0 + b1 + b2).reshape(1, H)
    wf_t = Wf.T.astype(jnp.bfloat16)
    bf2 = bf.reshape(1, H)

    grid = n_nodes // NODE_BLK
    out = pl.pallas_call(
        _tc_block,
        grid=(grid,),
        in_specs=[
            pl.BlockSpec((CS, NODE_BLK, H2), lambda i: (0, i, 0)),
            pl.BlockSpec((3 * H2, H), lambda i: (0, 0)),
            pl.BlockSpec((3 * H2, H), lambda i: (0, 0)),
            pl.BlockSpec((1, H), lambda i: (0, 0)),
            pl.BlockSpec((H, H), lambda i: (0, 0)),
            pl.BlockSpec((1, H), lambda i: (0, 0)),
        ],
        out_specs=pl.BlockSpec((NODE_BLK, OUT_C, H), lambda i: (i, 0, 0)),
        out_shape=jax.ShapeDtypeStruct((n_nodes, OUT_C, H), jnp.float32),
    )(xg3, ws_lo, ws_hi, bs, wf_t, bf2)

    padding_size = jnp.sum(ci == -1, axis=-1)
    num_neighbors = jnp.where(padding_size == CS, 0, CS - padding_size - 2)
    msg_mask = jnp.arange(1, OUT_C + 1)[None, :] <= num_neighbors[:, None]
    return out, msg_mask


# speedup vs baseline: 1.0197x; 1.0197x over previous
import functools

import jax
import jax.numpy as jnp
from jax import lax
from jax.experimental import pallas as pl
from jax.experimental.pallas import tpu as pltpu
from jax.experimental.pallas import tpu_sc as plsc

H = 256
CS = 18
OUT_C = 16

NP = 51200
ROWS = NP * CS
NW = 32
CHUNK = 128
N_CHUNKS_TOT = ROWS // CHUNK
DEPTH = 4
Q_BASE = N_CHUNKS_TOT // DEPTH // NW
Q_EXTRA = N_CHUNKS_TOT // DEPTH - Q_BASE * NW
MAX_CHUNKS = (Q_BASE + 1) * DEPTH
SLAB = ((MAX_CHUNKS + 7) // 8 + 1) * 8
IDX_ROWS = ((N_CHUNKS_TOT + SLAB + 7) // 8) * 8

NODE_BLK = 400


def _sc_gather_body(x_hbm, idx_hbm, out_hbm, idx_v,
                    buf0, buf1, buf2, buf3,
                    sg0, sg1, sg2, sg3, sw0, sw1, sw2, sw3):
    c = lax.axis_index("c")
    s = lax.axis_index("s")
    wid = s * 2 + c
    start = wid * Q_BASE * DEPTH + DEPTH * jnp.minimum(wid, Q_EXTRA)
    nq = jnp.where(wid < Q_EXTRA, Q_BASE + 1, Q_BASE)
    bufs = (buf0, buf1, buf2, buf3)
    gsems = (sg0, sg1, sg2, sg3)
    wsems = (sw0, sw1, sw2, sw3)
    astart = pl.multiple_of((start // 8) * 8, 8)
    off = start - astart

    def gather(k, i):
        pltpu.async_copy(x_hbm.at[idx_v.at[off + k]], bufs[i], gsems[i])

    def write(k, i):
        pltpu.async_copy(
            bufs[i], out_hbm.at[pl.ds((start + k) * CHUNK, CHUNK)], wsems[i])

    def wait_g(k, i):
        pltpu.make_async_copy(
            x_hbm.at[idx_v.at[off + k]], bufs[i], gsems[i]).wait()

    def wait_w(k, i):
        pltpu.make_async_copy(
            bufs[i], out_hbm.at[pl.ds((start + k) * CHUNK, CHUNK)],
            wsems[i]).wait()

    pltpu.sync_copy(idx_hbm.at[pl.ds(astart, SLAB)], idx_v)
    for i in range(DEPTH):
        gather(i, i)

    def body(j, carry):
        base = j * DEPTH
        for i in range(DEPTH):
            wait_g(base + i, i)
            write(base + i, i)
        for i in range(DEPTH):
            wait_w(base + i, i)
            gather(base + DEPTH + i, i)
        return carry

    lax.fori_loop(0, nq - 1, body, 0)
    last = (nq - 1) * DEPTH
    for i in range(DEPTH):
        wait_g(last + i, i)
        write(last + i, i)
    for i in range(DEPTH):
        wait_w(last + i, i)


H2 = H // 2

_sc_gather = functools.partial(
    pl.kernel,
    mesh=plsc.VectorSubcoreMesh(core_axis_name="c", subcore_axis_name="s"),
    out_type=jax.ShapeDtypeStruct((ROWS, H2), jnp.float32),
    scratch_types=(
        [pltpu.VMEM((SLAB, CHUNK), jnp.int32)]
        + [pltpu.VMEM((CHUNK, H2), jnp.float32) for _ in range(DEPTH)]
        + [pltpu.SemaphoreType.DMA for _ in range(2 * DEPTH)]
    ),
)(_sc_gather_body)


def _tc_block(xg_ref, wlo_ref, whi_ref, bs_ref, wf_ref, bf_ref, out_ref):
    u = jax.lax.bitcast_convert_type(xg_ref[...], jnp.uint32)
    xlo = jax.lax.bitcast_convert_type(
        u << 16, jnp.float32).astype(jnp.bfloat16)
    xhi = jax.lax.bitcast_convert_type(
        u & jnp.uint32(0xFFFF0000), jnp.float32).astype(jnp.bfloat16)
    for c in range(OUT_C):
        ylo = jnp.concatenate([xlo[c], xlo[c + 1], xlo[c + 2]], axis=1)
        yhi = jnp.concatenate([xhi[c], xhi[c + 1], xhi[c + 2]], axis=1)
        s = jnp.dot(ylo, wlo_ref[...], preferred_element_type=jnp.float32)
        s = s + jnp.dot(yhi, whi_ref[...], preferred_element_type=jnp.float32)
        s = s + bs_ref[...]
        e = jnp.where(s > 0, s, jnp.exp(jnp.minimum(s, 0.0)) - 1.0)
        o = jnp.dot(e.astype(jnp.bfloat16), wf_ref[...],
                    preferred_element_type=jnp.float32)
        out_ref[:, c, :] = o + bf_ref[...]


def kernel(x, circle_index, W0, b0, W1, b1, W2, b2, Wf, bf):
    n_nodes = x.shape[0]
    ci = circle_index.astype(jnp.int32)
    cit = jnp.pad(ci.T, ((0, 0), (0, NP - n_nodes)))
    idx = jnp.pad(cit.reshape(N_CHUNKS_TOT, CHUNK),
                  ((0, IDX_ROWS - N_CHUNKS_TOT), (0, 0)))

    xp = jax.lax.bitcast_convert_type(
        x.astype(jnp.bfloat16).reshape(n_nodes, H2, 2), jnp.float32)

    xg = _sc_gather(xp, idx)
    xg3 = xg.reshape(CS, NP, H2)

    ws = jnp.concatenate([W0.T, W1.T, W2.T], axis=0)
    ws = ws.reshape(3, H2, 2, H).astype(jnp.bfloat16)
    ws_lo = ws[:, :, 0, :].reshape(3 * H2, H)
    ws_hi = ws[:, :, 1, :].reshape(3 * H2, H)
    bs = (b0 + b1 + b2).reshape(1, H)
    wf_t = Wf.T.astype(jnp.bfloat16)
    bf2 = bf.reshape(1, H)

    grid = n_nodes // NODE_BLK
    out = pl.pallas_call(
        _tc_block,
        grid=(grid,),
        in_specs=[
            pl.BlockSpec((CS, NODE_BLK, H2), lambda i: (0, i, 0)),
            pl.BlockSpec((3 * H2, H), lambda i: (0, 0)),
            pl.BlockSpec((3 * H2, H), lambda i: (0, 0)),
            pl.BlockSpec((1, H), lambda i: (0, 0)),
            pl.BlockSpec((H, H), lambda i: (0, 0)),
            pl.BlockSpec((1, H), lambda i: (0, 0)),
        ],
        out_specs=pl.BlockSpec((NODE_BLK, OUT_C, H), lambda i: (i, 0, 0)),
        out_shape=jax.ShapeDtypeStruct((n_nodes, OUT_C, H), jnp.float32),
    )(xg3, ws_lo, ws_hi, bs, wf_t, bf2)

    padding_size = jnp.sum(ci == -1, axis=-1)
    num_neighbors = jnp.where(padding_size == CS, 0, CS - padding_size - 2)
    msg_mask = jnp.arange(1, OUT_C + 1)[None, :] <= num_neighbors[:, None]
    return out, msg_mask


# speedup vs baseline: 1.1700x; 1.1474x over previous
import functools

import jax
import jax.numpy as jnp
from jax import lax
from jax.experimental import pallas as pl
from jax.experimental.pallas import tpu as pltpu
from jax.experimental.pallas import tpu_sc as plsc

H = 256
CS = 18
OUT_C = 16

NP = 51200
ROWS = NP * CS
NW = 32
CHUNK = 128
N_CHUNKS_TOT = ROWS // CHUNK
DEPTH = 4
Q_BASE = N_CHUNKS_TOT // DEPTH // NW
Q_EXTRA = N_CHUNKS_TOT // DEPTH - Q_BASE * NW
MAX_CHUNKS = (Q_BASE + 1) * DEPTH
SLAB = ((MAX_CHUNKS + 7) // 8 + 1) * 8
IDX_ROWS = ((N_CHUNKS_TOT + SLAB + 7) // 8) * 8

NODE_BLK = 400


def _sc_gather_body(x_hbm, idx_hbm, out_hbm, idx_v,
                    buf0, buf1, buf2, buf3,
                    sg0, sg1, sg2, sg3, sw0, sw1, sw2, sw3):
    c = lax.axis_index("c")
    s = lax.axis_index("s")
    wid = s * 2 + c
    start = wid * Q_BASE * DEPTH + DEPTH * jnp.minimum(wid, Q_EXTRA)
    nq = jnp.where(wid < Q_EXTRA, Q_BASE + 1, Q_BASE)
    bufs = (buf0, buf1, buf2, buf3)
    gsems = (sg0, sg1, sg2, sg3)
    wsems = (sw0, sw1, sw2, sw3)
    astart = pl.multiple_of((start // 8) * 8, 8)
    off = start - astart

    def gather(k, i):
        pltpu.async_copy(x_hbm.at[idx_v.at[off + k]], bufs[i], gsems[i])

    def write(k, i):
        pltpu.async_copy(
            bufs[i], out_hbm.at[pl.ds((start + k) * CHUNK, CHUNK)], wsems[i])

    def wait_g(k, i):
        pltpu.make_async_copy(
            x_hbm.at[idx_v.at[off + k]], bufs[i], gsems[i]).wait()

    def wait_w(k, i):
        pltpu.make_async_copy(
            bufs[i], out_hbm.at[pl.ds((start + k) * CHUNK, CHUNK)],
            wsems[i]).wait()

    pltpu.sync_copy(idx_hbm.at[pl.ds(astart, SLAB)], idx_v)
    for i in range(DEPTH):
        gather(i, i)

    def body(j, carry):
        base = j * DEPTH
        for i in range(DEPTH):
            wait_g(base + i, i)
            write(base + i, i)
        for i in range(DEPTH):
            wait_w(base + i, i)
            gather(base + DEPTH + i, i)
        return carry

    lax.fori_loop(0, nq - 1, body, 0)
    last = (nq - 1) * DEPTH
    for i in range(DEPTH):
        wait_g(last + i, i)
        write(last + i, i)
    for i in range(DEPTH):
        wait_w(last + i, i)


H2 = H // 2

_sc_gather = functools.partial(
    pl.kernel,
    mesh=plsc.VectorSubcoreMesh(core_axis_name="c", subcore_axis_name="s"),
    out_type=jax.ShapeDtypeStruct((ROWS, H2), jnp.float32),
    scratch_types=(
        [pltpu.VMEM((SLAB, CHUNK), jnp.int32)]
        + [pltpu.VMEM((CHUNK, H2), jnp.float32) for _ in range(DEPTH)]
        + [pltpu.SemaphoreType.DMA for _ in range(2 * DEPTH)]
    ),
)(_sc_gather_body)


def _tc_block(xg_ref, wlo_ref, whi_ref, bs_ref, wf_ref, bf_ref, out_ref):
    u = jax.lax.bitcast_convert_type(xg_ref[...], jnp.uint32)
    xlo = jax.lax.bitcast_convert_type(
        u << 16, jnp.float32).astype(jnp.bfloat16)
    xhi = jax.lax.bitcast_convert_type(
        u & jnp.uint32(0xFFFF0000), jnp.float32).astype(jnp.bfloat16)
    for c in range(OUT_C):
        ylo = jnp.concatenate([xlo[c], xlo[c + 1], xlo[c + 2]], axis=1)
        yhi = jnp.concatenate([xhi[c], xhi[c + 1], xhi[c + 2]], axis=1)
        s = jnp.dot(ylo, wlo_ref[...], preferred_element_type=jnp.float32)
        s = s + jnp.dot(yhi, whi_ref[...], preferred_element_type=jnp.float32)
        s = s + bs_ref[...]
        e = jnp.where(s > 0, s, jnp.exp(jnp.minimum(s, 0.0)) - 1.0)
        o = jnp.dot(e.astype(jnp.bfloat16), wf_ref[...],
                    preferred_element_type=jnp.float32)
        out_ref[:, c, :] = o + bf_ref[...]


def kernel(x, circle_index, W0, b0, W1, b1, W2, b2, Wf, bf):
    n_nodes = x.shape[0]
    ci = circle_index.astype(jnp.int32)
    cit = jnp.pad(ci.T, ((0, 0), (0, NP - n_nodes)))
    idx = jnp.pad(cit.reshape(N_CHUNKS_TOT, CHUNK),
                  ((0, IDX_ROWS - N_CHUNKS_TOT), (0, 0)))

    lo16 = jax.lax.bitcast_convert_type(
        x[:, :H2].astype(jnp.bfloat16), jnp.uint16).astype(jnp.uint32)
    hi16 = jax.lax.bitcast_convert_type(
        x[:, H2:].astype(jnp.bfloat16), jnp.uint16).astype(jnp.uint32)
    xp = jax.lax.bitcast_convert_type(lo16 | (hi16 << 16), jnp.float32)

    xg = _sc_gather(xp, idx)
    xg3 = xg.reshape(CS, NP, H2)

    ws = jnp.concatenate([W0.T, W1.T, W2.T], axis=0)
    ws = ws.reshape(3, 2, H2, H).astype(jnp.bfloat16)
    ws_lo = ws[:, 0].reshape(3 * H2, H)
    ws_hi = ws[:, 1].reshape(3 * H2, H)
    bs = (b0 + b1 + b2).reshape(1, H)
    wf_t = Wf.T.astype(jnp.bfloat16)
    bf2 = bf.reshape(1, H)

    grid = n_nodes // NODE_BLK
    out = pl.pallas_call(
        _tc_block,
        grid=(grid,),
        in_specs=[
            pl.BlockSpec((CS, NODE_BLK, H2), lambda i: (0, i, 0)),
            pl.BlockSpec((3 * H2, H), lambda i: (0, 0)),
            pl.BlockSpec((3 * H2, H), lambda i: (0, 0)),
            pl.BlockSpec((1, H), lambda i: (0, 0)),
            pl.BlockSpec((H, H), lambda i: (0, 0)),
            pl.BlockSpec((1, H), lambda i: (0, 0)),
        ],
        out_specs=pl.BlockSpec((NODE_BLK, OUT_C, H), lambda i: (i, 0, 0)),
        out_shape=jax.ShapeDtypeStruct((n_nodes, OUT_C, H), jnp.float32),
    )(xg3, ws_lo, ws_hi, bs, wf_t, bf2)

    padding_size = jnp.sum(ci == -1, axis=-1)
    num_neighbors = jnp.where(padding_size == CS, 0, CS - padding_size - 2)
    msg_mask = jnp.arange(1, OUT_C + 1)[None, :] <= num_neighbors[:, None]
    return out, msg_mask


# speedup vs baseline: 1.7131x; 1.4642x over previous
import functools

import jax
import jax.numpy as jnp
from jax import lax
from jax.experimental import pallas as pl
from jax.experimental.pallas import tpu as pltpu
from jax.experimental.pallas import tpu_sc as plsc

H = 256
CS = 18
OUT_C = 16

NSL = 4
NP = 51200
NP_S = NP // NSL
ROWS_S = NP_S * CS
NW = 32
CHUNK = 128
N_CHUNKS_TOT = ROWS_S // CHUNK
DEPTH = 4
Q_BASE = N_CHUNKS_TOT // DEPTH // NW
Q_EXTRA = N_CHUNKS_TOT // DEPTH - Q_BASE * NW
MAX_CHUNKS = (Q_BASE + 1) * DEPTH
SLAB = ((MAX_CHUNKS + 7) // 8 + 1) * 8
IDX_ROWS = ((N_CHUNKS_TOT + SLAB + 7) // 8) * 8

NODE_BLK = 400


def _sc_gather_body(x_hbm, idx_hbm, out_hbm, idx_v,
                    buf0, buf1, buf2, buf3,
                    sg0, sg1, sg2, sg3, sw0, sw1, sw2, sw3):
    c = lax.axis_index("c")
    s = lax.axis_index("s")
    wid = s * 2 + c
    start = wid * Q_BASE * DEPTH + DEPTH * jnp.minimum(wid, Q_EXTRA)
    nq = jnp.where(wid < Q_EXTRA, Q_BASE + 1, Q_BASE)
    bufs = (buf0, buf1, buf2, buf3)
    gsems = (sg0, sg1, sg2, sg3)
    wsems = (sw0, sw1, sw2, sw3)
    astart = pl.multiple_of((start // 8) * 8, 8)
    off = start - astart

    def gather(k, i):
        pltpu.async_copy(x_hbm.at[idx_v.at[off + k]], bufs[i], gsems[i])

    def write(k, i):
        pltpu.async_copy(
            bufs[i], out_hbm.at[pl.ds((start + k) * CHUNK, CHUNK)], wsems[i])

    def wait_g(k, i):
        pltpu.make_async_copy(
            x_hbm.at[idx_v.at[off + k]], bufs[i], gsems[i]).wait()

    def wait_w(k, i):
        pltpu.make_async_copy(
            bufs[i], out_hbm.at[pl.ds((start + k) * CHUNK, CHUNK)],
            wsems[i]).wait()

    pltpu.sync_copy(idx_hbm.at[pl.ds(astart, SLAB)], idx_v)
    for i in range(DEPTH):
        gather(i, i)

    def body(j, carry):
        base = j * DEPTH
        for i in range(DEPTH):
            wait_g(base + i, i)
            write(base + i, i)
        for i in range(DEPTH):
            wait_w(base + i, i)
            gather(base + DEPTH + i, i)
        return carry

    lax.fori_loop(0, nq - 1, body, 0)
    last = (nq - 1) * DEPTH
    for i in range(DEPTH):
        wait_g(last + i, i)
        write(last + i, i)
    for i in range(DEPTH):
        wait_w(last + i, i)


H2 = H // 2

_sc_gather = functools.partial(
    pl.kernel,
    mesh=plsc.VectorSubcoreMesh(core_axis_name="c", subcore_axis_name="s"),
    out_type=jax.ShapeDtypeStruct((ROWS_S, H2), jnp.float32),
    scratch_types=(
        [pltpu.VMEM((SLAB, CHUNK), jnp.int32)]
        + [pltpu.VMEM((CHUNK, H2), jnp.float32) for _ in range(DEPTH)]
        + [pltpu.SemaphoreType.DMA for _ in range(2 * DEPTH)]
    ),
)(_sc_gather_body)


def _tc_block(xg_ref, wlo_ref, whi_ref, bs_ref, wf_ref, bf_ref, *rest):
    out_ref = rest[-1]
    u = jax.lax.bitcast_convert_type(xg_ref[...], jnp.uint32)
    xlo = jax.lax.bitcast_convert_type(
        u << 16, jnp.float32).astype(jnp.bfloat16)
    xhi = jax.lax.bitcast_convert_type(
        u & jnp.uint32(0xFFFF0000), jnp.float32).astype(jnp.bfloat16)
    for c in range(OUT_C):
        ylo = jnp.concatenate([xlo[c], xlo[c + 1], xlo[c + 2]], axis=1)
        yhi = jnp.concatenate([xhi[c], xhi[c + 1], xhi[c + 2]], axis=1)
        s = jnp.dot(ylo, wlo_ref[...], preferred_element_type=jnp.float32)
        s = s + jnp.dot(yhi, whi_ref[...], preferred_element_type=jnp.float32)
        s = s + bs_ref[...]
        e = jnp.where(s > 0, s, jnp.exp(jnp.minimum(s, 0.0)) - 1.0)
        o = jnp.dot(e.astype(jnp.bfloat16), wf_ref[...],
                    preferred_element_type=jnp.float32)
        out_ref[:, c, :] = o + bf_ref[...]


def kernel(x, circle_index, W0, b0, W1, b1, W2, b2, Wf, bf):
    n_nodes = x.shape[0]
    ci = circle_index.astype(jnp.int32)
    cit = jnp.pad(ci.T, ((0, 0), (0, NP - n_nodes)))

    lo16 = jax.lax.bitcast_convert_type(
        x[:, :H2].astype(jnp.bfloat16), jnp.uint16).astype(jnp.uint32)
    hi16 = jax.lax.bitcast_convert_type(
        x[:, H2:].astype(jnp.bfloat16), jnp.uint16).astype(jnp.uint32)
    xp = jax.lax.bitcast_convert_type(lo16 | (hi16 << 16), jnp.float32)

    ws = jnp.concatenate([W0.T, W1.T, W2.T], axis=0)
    ws = ws.reshape(3, 2, H2, H).astype(jnp.bfloat16)
    ws_lo = ws[:, 0].reshape(3 * H2, H)
    ws_hi = ws[:, 1].reshape(3 * H2, H)
    bs = (b0 + b1 + b2).reshape(1, H)
    wf_t = Wf.T.astype(jnp.bfloat16)
    bf2 = bf.reshape(1, H)

    nblk_total = n_nodes // NODE_BLK
    nblk_slice = NP_S // NODE_BLK
    out = None
    for b in range(NSL):
        cit_b = cit[:, b * NP_S:(b + 1) * NP_S]
        idx_b = jnp.pad(cit_b.reshape(N_CHUNKS_TOT, CHUNK),
                        ((0, IDX_ROWS - N_CHUNKS_TOT), (0, 0)))
        xg3 = _sc_gather(xp, idx_b).reshape(CS, NP_S, H2)

        boff = b * nblk_slice
        grid_b = min(nblk_slice, nblk_total - boff)
        in_specs = [
            pl.BlockSpec((CS, NODE_BLK, H2), lambda i: (0, i, 0)),
            pl.BlockSpec((3 * H2, H), lambda i: (0, 0)),
            pl.BlockSpec((3 * H2, H), lambda i: (0, 0)),
            pl.BlockSpec((1, H), lambda i: (0, 0)),
            pl.BlockSpec((H, H), lambda i: (0, 0)),
            pl.BlockSpec((1, H), lambda i: (0, 0)),
        ]
        args = [xg3, ws_lo, ws_hi, bs, wf_t, bf2]
        aliases = {}
        if out is not None:
            in_specs.append(pl.BlockSpec(memory_space=pl.ANY))
            args.append(out)
            aliases = {6: 0}
        out = pl.pallas_call(
            _tc_block,
            grid=(grid_b,),
            in_specs=in_specs,
            out_specs=pl.BlockSpec(
                (NODE_BLK, OUT_C, H),
                lambda i, _boff=boff: (_boff + i, 0, 0)),
            out_shape=jax.ShapeDtypeStruct((n_nodes, OUT_C, H), jnp.float32),
            input_output_aliases=aliases,
        )(*args)

    padding_size = jnp.sum(ci == -1, axis=-1)
    num_neighbors = jnp.where(padding_size == CS, 0, CS - padding_size - 2)
    msg_mask = jnp.arange(1, OUT_C + 1)[None, :] <= num_neighbors[:, None]
    return out, msg_mask
